# Initial kernel scaffold; baseline (speedup 1.0000x reference)
import jax
import jax.numpy as jnp
from jax.experimental import pallas as pl


def kernel(in_points_list, in_batch_list, out_points_list, out_batch_list, mean, variance):
    raise NotImplementedError("write your pallas kernel here")


# speedup vs baseline: 46.0278x; 46.0278x over previous
import functools

import jax
import jax.numpy as jnp
from jax import lax
from jax.experimental import pallas as pl
from jax.experimental.pallas import tpu as pltpu
from jax.experimental.pallas import tpu_sc as plsc

ALFA_HALF = 0.25
L = 16
NW = 32


def _make_sc_chamfer(n_points, n_kl, interpret=False):
    ppw = n_points // NW
    n_chunks = ppw // L
    klw = n_kl // NW
    mesh = plsc.VectorSubcoreMesh(core_axis_name="c", subcore_axis_name="s",
                                  num_cores=2, num_subcores=16)
    INF = jnp.float32(jnp.inf)

    @functools.partial(
        pl.kernel,
        out_type=jax.ShapeDtypeStruct((NW * L,), jnp.float32),
        mesh=mesh,
        interpret=interpret,
        compiler_params=pltpu.CompilerParams(needs_layout_passes=False),
        scratch_types=[
            pltpu.VMEM((n_points,), jnp.float32),
            pltpu.VMEM((n_points,), jnp.float32),
            pltpu.VMEM((n_points,), jnp.float32),
            pltpu.VMEM((ppw,), jnp.float32),
            pltpu.VMEM((ppw,), jnp.float32),
            pltpu.VMEM((ppw,), jnp.float32),
            pltpu.VMEM((ppw,), jnp.int32),
            pltpu.VMEM((L,), jnp.int32),
            pltpu.VMEM((L,), jnp.int32),
            pltpu.VMEM((klw,), jnp.float32),
            pltpu.VMEM((klw,), jnp.float32),
            pltpu.VMEM((L,), jnp.float32),
        ],
    )
    def chamfer_kernel(x1, y1, z1, b1, x2, y2, z2, b2, aux, mv, vv, out,
                       cx, cy, cz, qx, qy, qz, qb, scs, nbs, klm, klv, accb):
        wid = lax.axis_index("s") * 2 + lax.axis_index("c")
        qbase = wid * ppw

        pltpu.sync_copy(aux.at[pl.ds(2 * L, L)], nbs)
        nbv = nbs[...]

        def direction(q_xyz, q_b, c_xyz, aux_row, accv):
            pltpu.sync_copy(c_xyz[0], cx)
            pltpu.sync_copy(c_xyz[1], cy)
            pltpu.sync_copy(c_xyz[2], cz)
            pltpu.sync_copy(q_xyz[0].at[pl.ds(qbase, ppw)], qx)
            pltpu.sync_copy(q_xyz[1].at[pl.ds(qbase, ppw)], qy)
            pltpu.sync_copy(q_xyz[2].at[pl.ds(qbase, ppw)], qz)
            pltpu.sync_copy(q_b.at[pl.ds(qbase, ppw)], qb)
            pltpu.sync_copy(aux.at[pl.ds(aux_row * L, L)], scs)

            zidx = jnp.zeros((L,), jnp.int32)
            cx0 = plsc.load_gather(cx, [zidx])
            cy0 = plsc.load_gather(cy, [zidx])
            cz0 = plsc.load_gather(cz, [zidx])

            def chunk_body(c, accv):
                qoff = c * L
                qxv = qx[pl.ds(qoff, L)]
                qyv = qy[pl.ds(qoff, L)]
                qzv = qz[pl.ds(qoff, L)]
                qbv = qb[pl.ds(qoff, L)]
                lov = plsc.load_gather(scs, [qbv])
                hiv = plsc.load_gather(scs, [qbv + 1])
                lo = jnp.min(lov)
                hi = jnp.max(hiv)
                j0 = lax.div(lo, L)
                j1 = lax.div(hi + (L - 1), L)

                def block_body(j, minv):
                    jg0 = j * L
                    for jj in range(L):
                        jg = jg0 + jj
                        idxv = jnp.full((L,), jg, jnp.int32)
                        cxs = plsc.load_gather(cx, [idxv])
                        cys = plsc.load_gather(cy, [idxv])
                        czs = plsc.load_gather(cz, [idxv])
                        dx = qxv - cxs
                        dy = qyv - cys
                        dz = qzv - czs
                        dd = dx * dx + dy * dy + dz * dz
                        valid = (jg >= lov) & (jg < hiv)
                        minv = jnp.where(valid, jnp.minimum(minv, dd), minv)
                    return minv

                minv = lax.fori_loop(j0, j1, block_body, jnp.full((L,), INF))
                fdx = qxv - cx0
                fdy = qyv - cy0
                fdz = qzv - cz0
                fb = fdx * fdx + fdy * fdy + fdz * fdz
                mfix = jnp.where(minv == INF, fb, minv)
                return accv + jnp.where(qbv < nbv, mfix, jnp.float32(0.0))

            return lax.fori_loop(0, n_chunks, chunk_body, accv)

        accv = jnp.zeros((L,), jnp.float32)
        accv = direction((x1, y1, z1), b1, (x2, y2, z2), 1, accv)
        accv = direction((x2, y2, z2), b2, (x1, y1, z1), 0, accv)

        pltpu.sync_copy(mv.at[pl.ds(wid * klw, klw)], klm)
        pltpu.sync_copy(vv.at[pl.ds(wid * klw, klw)], klv)
        for t in range(klw // L):
            mb = klm[pl.ds(t * L, L)]
            vb = klv[pl.ds(t * L, L)]
            accv = accv + jnp.float32(ALFA_HALF) * (jnp.exp(vb) + mb * mb - vb)

        accb[...] = accv
        pltpu.sync_copy(accb, out.at[pl.ds(wid * L, L)])

    return chamfer_kernel


def _run(p1, b1, p2, b2, mean, variance, interpret=False):
    n = p1.shape[0]
    b1 = b1.astype(jnp.int32)
    b2 = b2.astype(jnp.int32)
    vals = jnp.arange(9, dtype=jnp.int32)
    s1 = jnp.searchsorted(b1, vals).astype(jnp.int32)
    s2 = jnp.searchsorted(b2, vals).astype(jnp.int32)
    pad = jnp.full((L - 9,), n, jnp.int32)
    nbrow = jnp.full((L,), b1[-1], jnp.int32)
    aux = jnp.concatenate([s1, pad, s2, pad, nbrow])
    mv = mean.reshape(-1)
    vv = variance.reshape(-1)
    fn = _make_sc_chamfer(n, mv.shape[0], interpret=interpret)
    partial = fn(p1[:, 0], p1[:, 1], p1[:, 2], b1,
                 p2[:, 0], p2[:, 1], p2[:, 2], b2, aux, mv, vv)
    return jnp.sum(partial)


def kernel(in_points_list, in_batch_list, out_points_list, out_batch_list,
           mean, variance):
    return _run(in_points_list[0], in_batch_list[0],
                out_points_list[0], out_batch_list[0], mean, variance)


# speedup vs baseline: 52.1810x; 1.1337x over previous
import functools

import jax
import jax.numpy as jnp
from jax import lax
from jax.experimental import pallas as pl
from jax.experimental.pallas import tpu as pltpu
from jax.experimental.pallas import tpu_sc as plsc

ALFA_HALF = 0.25
L = 16
NW = 32


def _make_sc_chamfer(n_points, n_kl, interpret=False):
    ppw = n_points // NW
    n_chunks = ppw // L
    klw = n_kl // NW
    mesh = plsc.VectorSubcoreMesh(core_axis_name="c", subcore_axis_name="s",
                                  num_cores=2, num_subcores=16)
    INF = jnp.float32(jnp.inf)

    @functools.partial(
        pl.kernel,
        out_type=jax.ShapeDtypeStruct((NW * L,), jnp.float32),
        mesh=mesh,
        interpret=interpret,
        compiler_params=pltpu.CompilerParams(needs_layout_passes=False),
        scratch_types=[
            pltpu.VMEM((n_points,), jnp.float32),
            pltpu.VMEM((n_points,), jnp.float32),
            pltpu.VMEM((n_points,), jnp.float32),
            pltpu.VMEM((ppw,), jnp.float32),
            pltpu.VMEM((ppw,), jnp.float32),
            pltpu.VMEM((ppw,), jnp.float32),
            pltpu.VMEM((ppw,), jnp.int32),
            pltpu.VMEM((L,), jnp.int32),
            pltpu.VMEM((L,), jnp.int32),
            pltpu.VMEM((klw,), jnp.float32),
            pltpu.VMEM((klw,), jnp.float32),
            pltpu.VMEM((L,), jnp.float32),
        ],
    )
    def chamfer_kernel(x1, y1, z1, b1, x2, y2, z2, b2, aux, mv, vv, out,
                       cx, cy, cz, qx, qy, qz, qb, scs, nbs, klm, klv, accb):
        wid = lax.axis_index("s") * 2 + lax.axis_index("c")
        qbase = wid * ppw

        pltpu.sync_copy(aux.at[pl.ds(2 * L, L)], nbs)
        nbv = nbs[...]

        def direction(q_xyz, q_b, c_xyz, aux_row, accv):
            pltpu.sync_copy(c_xyz[0], cx)
            pltpu.sync_copy(c_xyz[1], cy)
            pltpu.sync_copy(c_xyz[2], cz)
            pltpu.sync_copy(q_xyz[0].at[pl.ds(qbase, ppw)], qx)
            pltpu.sync_copy(q_xyz[1].at[pl.ds(qbase, ppw)], qy)
            pltpu.sync_copy(q_xyz[2].at[pl.ds(qbase, ppw)], qz)
            pltpu.sync_copy(q_b.at[pl.ds(qbase, ppw)], qb)
            pltpu.sync_copy(aux.at[pl.ds(aux_row * L, L)], scs)

            zidx = jnp.zeros((L,), jnp.int32)
            cx0 = plsc.load_gather(cx, [zidx])
            cy0 = plsc.load_gather(cy, [zidx])
            cz0 = plsc.load_gather(cz, [zidx])

            def chunk_body(c, accv):
                qoff = c * L
                qxv = qx[pl.ds(qoff, L)]
                qyv = qy[pl.ds(qoff, L)]
                qzv = qz[pl.ds(qoff, L)]
                qbv = qb[pl.ds(qoff, L)]
                lov = plsc.load_gather(scs, [qbv])
                hiv = plsc.load_gather(scs, [qbv + 1])
                lo = jnp.min(lov)
                hi = jnp.max(hiv)
                j0 = lax.div(lo, L)
                j1 = lax.div(hi + (L - 1), L)
                ja = jnp.minimum(lax.div(jnp.max(lov) + (L - 1), L), j1)
                jb = jnp.maximum(lax.div(jnp.min(hiv), L), ja)

                def masked_body(j, minv):
                    jg0 = j * L
                    for jj in range(L):
                        jg = jg0 + jj
                        idxv = jnp.full((L,), jg, jnp.int32)
                        cxs = plsc.load_gather(cx, [idxv])
                        cys = plsc.load_gather(cy, [idxv])
                        czs = plsc.load_gather(cz, [idxv])
                        dx = qxv - cxs
                        dy = qyv - cys
                        dz = qzv - czs
                        dd = dx * dx + dy * dy + dz * dz
                        valid = (jg >= lov) & (jg < hiv)
                        minv = jnp.where(valid, jnp.minimum(minv, dd), minv)
                    return minv

                def inner_body(j, minv):
                    jg0 = j * L
                    for jj in range(L):
                        idxv = jnp.full((L,), jg0 + jj, jnp.int32)
                        cxs = plsc.load_gather(cx, [idxv])
                        cys = plsc.load_gather(cy, [idxv])
                        czs = plsc.load_gather(cz, [idxv])
                        dx = qxv - cxs
                        dy = qyv - cys
                        dz = qzv - czs
                        dd = dx * dx + dy * dy + dz * dz
                        minv = jnp.minimum(minv, dd)
                    return minv

                minv = lax.fori_loop(j0, ja, masked_body, jnp.full((L,), INF))
                minv = lax.fori_loop(ja, jb, inner_body, minv)
                minv = lax.fori_loop(jb, j1, masked_body, minv)
                fdx = qxv - cx0
                fdy = qyv - cy0
                fdz = qzv - cz0
                fb = fdx * fdx + fdy * fdy + fdz * fdz
                mfix = jnp.where(minv == INF, fb, minv)
                return accv + jnp.where(qbv < nbv, mfix, jnp.float32(0.0))

            return lax.fori_loop(0, n_chunks, chunk_body, accv)

        accv = jnp.zeros((L,), jnp.float32)
        accv = direction((x1, y1, z1), b1, (x2, y2, z2), 1, accv)
        accv = direction((x2, y2, z2), b2, (x1, y1, z1), 0, accv)

        pltpu.sync_copy(mv.at[pl.ds(wid * klw, klw)], klm)
        pltpu.sync_copy(vv.at[pl.ds(wid * klw, klw)], klv)
        for t in range(klw // L):
            mb = klm[pl.ds(t * L, L)]
            vb = klv[pl.ds(t * L, L)]
            accv = accv + jnp.float32(ALFA_HALF) * (jnp.exp(vb) + mb * mb - vb)

        accb[...] = accv
        pltpu.sync_copy(accb, out.at[pl.ds(wid * L, L)])

    return chamfer_kernel


def _run(p1, b1, p2, b2, mean, variance, interpret=False):
    n = p1.shape[0]
    b1 = b1.astype(jnp.int32)
    b2 = b2.astype(jnp.int32)
    vals = jnp.arange(9, dtype=jnp.int32)
    s1 = jnp.searchsorted(b1, vals).astype(jnp.int32)
    s2 = jnp.searchsorted(b2, vals).astype(jnp.int32)
    pad = jnp.full((L - 9,), n, jnp.int32)
    nbrow = jnp.full((L,), b1[-1], jnp.int32)
    aux = jnp.concatenate([s1, pad, s2, pad, nbrow])
    mv = mean.reshape(-1)
    vv = variance.reshape(-1)
    fn = _make_sc_chamfer(n, mv.shape[0], interpret=interpret)
    partial = fn(p1[:, 0], p1[:, 1], p1[:, 2], b1,
                 p2[:, 0], p2[:, 1], p2[:, 2], b2, aux, mv, vv)
    return jnp.sum(partial)


def kernel(in_points_list, in_batch_list, out_points_list, out_batch_list,
           mean, variance):
    return _run(in_points_list[0], in_batch_list[0],
                out_points_list[0], out_batch_list[0], mean, variance)


# speedup vs baseline: 53.4161x; 1.0237x over previous
import functools

import jax
import jax.numpy as jnp
from jax import lax
from jax.experimental import pallas as pl
from jax.experimental.pallas import tpu as pltpu
from jax.experimental.pallas import tpu_sc as plsc

ALFA_HALF = 0.25
L = 16
NW = 32


def _make_sc_chamfer(n_points, n_kl, interpret=False):
    ppw = n_points // NW
    n_chunks = ppw // L
    klw = n_kl // NW
    mesh = plsc.VectorSubcoreMesh(core_axis_name="c", subcore_axis_name="s",
                                  num_cores=2, num_subcores=16)
    INF = jnp.float32(jnp.inf)

    @functools.partial(
        pl.kernel,
        out_type=jax.ShapeDtypeStruct((NW * L,), jnp.float32),
        mesh=mesh,
        interpret=interpret,
        compiler_params=pltpu.CompilerParams(needs_layout_passes=False),
        scratch_types=[
            pltpu.VMEM((n_points,), jnp.float32),
            pltpu.VMEM((n_points,), jnp.float32),
            pltpu.VMEM((n_points,), jnp.float32),
            pltpu.VMEM((ppw,), jnp.float32),
            pltpu.VMEM((ppw,), jnp.float32),
            pltpu.VMEM((ppw,), jnp.float32),
            pltpu.VMEM((ppw,), jnp.int32),
            pltpu.VMEM((L,), jnp.int32),
            pltpu.VMEM((L,), jnp.int32),
            pltpu.VMEM((klw,), jnp.float32),
            pltpu.VMEM((klw,), jnp.float32),
            pltpu.VMEM((L,), jnp.float32),
        ],
    )
    def chamfer_kernel(x1, y1, z1, b1, x2, y2, z2, b2, aux, mv, vv, out,
                       cx, cy, cz, qx, qy, qz, qb, scs, nbs, klm, klv, accb):
        wid = lax.axis_index("s") * 2 + lax.axis_index("c")
        qbase = wid * ppw

        pltpu.sync_copy(aux.at[pl.ds(2 * L, L)], nbs)
        nbv = nbs[...]

        def direction(q_xyz, q_b, c_xyz, aux_row, accv):
            pltpu.sync_copy(c_xyz[0], cx)
            pltpu.sync_copy(c_xyz[1], cy)
            pltpu.sync_copy(c_xyz[2], cz)
            pltpu.sync_copy(q_xyz[0].at[pl.ds(qbase, ppw)], qx)
            pltpu.sync_copy(q_xyz[1].at[pl.ds(qbase, ppw)], qy)
            pltpu.sync_copy(q_xyz[2].at[pl.ds(qbase, ppw)], qz)
            pltpu.sync_copy(q_b.at[pl.ds(qbase, ppw)], qb)
            pltpu.sync_copy(aux.at[pl.ds(aux_row * L, L)], scs)

            zidx = jnp.zeros((L,), jnp.int32)
            cx0 = plsc.load_gather(cx, [zidx])
            cy0 = plsc.load_gather(cy, [zidx])
            cz0 = plsc.load_gather(cz, [zidx])

            def chunk_body(c, accv):
                qoff = c * L
                qxv = qx[pl.ds(qoff, L)]
                qyv = qy[pl.ds(qoff, L)]
                qzv = qz[pl.ds(qoff, L)]
                qbv = qb[pl.ds(qoff, L)]
                lov = plsc.load_gather(scs, [qbv])
                hiv = plsc.load_gather(scs, [qbv + 1])
                lo = jnp.min(lov)
                hi = jnp.max(hiv)
                j0 = lax.div(lo, L)
                j1 = lax.div(hi + (L - 1), L)
                ja = jnp.minimum(lax.div(jnp.max(lov) + (L - 1), L), j1)
                jb = jnp.maximum(lax.div(jnp.min(hiv), L), ja)

                def masked_body(j, minv):
                    jg0 = j * L
                    for jj in range(L):
                        jg = jg0 + jj
                        idxv = jnp.full((L,), jg, jnp.int32)
                        cxs = plsc.load_gather(cx, [idxv])
                        cys = plsc.load_gather(cy, [idxv])
                        czs = plsc.load_gather(cz, [idxv])
                        dx = qxv - cxs
                        dy = qyv - cys
                        dz = qzv - czs
                        dd = dx * dx + dy * dy + dz * dz
                        valid = (jg >= lov) & (jg < hiv)
                        minv = jnp.where(valid, jnp.minimum(minv, dd), minv)
                    return minv

                def inner_body(j, minv):
                    boff = j * L
                    cxb = cx[pl.ds(boff, L)]
                    cyb = cy[pl.ds(boff, L)]
                    czb = cz[pl.ds(boff, L)]
                    for jj in range(L):
                        lane = jnp.full((L,), jj, jnp.int32)
                        cxs = jnp.take_along_axis(
                            cxb, lane, axis=0, mode="promise_in_bounds")
                        cys = jnp.take_along_axis(
                            cyb, lane, axis=0, mode="promise_in_bounds")
                        czs = jnp.take_along_axis(
                            czb, lane, axis=0, mode="promise_in_bounds")
                        dx = qxv - cxs
                        dy = qyv - cys
                        dz = qzv - czs
                        dd = dx * dx + dy * dy + dz * dz
                        minv = jnp.minimum(minv, dd)
                    return minv

                minv = lax.fori_loop(j0, ja, masked_body, jnp.full((L,), INF))
                minv = lax.fori_loop(ja, jb, inner_body, minv)
                minv = lax.fori_loop(jb, j1, masked_body, minv)
                fdx = qxv - cx0
                fdy = qyv - cy0
                fdz = qzv - cz0
                fb = fdx * fdx + fdy * fdy + fdz * fdz
                mfix = jnp.where(minv == INF, fb, minv)
                return accv + jnp.where(qbv < nbv, mfix, jnp.float32(0.0))

            return lax.fori_loop(0, n_chunks, chunk_body, accv)

        accv = jnp.zeros((L,), jnp.float32)
        accv = direction((x1, y1, z1), b1, (x2, y2, z2), 1, accv)
        accv = direction((x2, y2, z2), b2, (x1, y1, z1), 0, accv)

        pltpu.sync_copy(mv.at[pl.ds(wid * klw, klw)], klm)
        pltpu.sync_copy(vv.at[pl.ds(wid * klw, klw)], klv)
        for t in range(klw // L):
            mb = klm[pl.ds(t * L, L)]
            vb = klv[pl.ds(t * L, L)]
            accv = accv + jnp.float32(ALFA_HALF) * (jnp.exp(vb) + mb * mb - vb)

        accb[...] = accv
        pltpu.sync_copy(accb, out.at[pl.ds(wid * L, L)])

    return chamfer_kernel


def _run(p1, b1, p2, b2, mean, variance, interpret=False):
    n = p1.shape[0]
    b1 = b1.astype(jnp.int32)
    b2 = b2.astype(jnp.int32)
    vals = jnp.arange(9, dtype=jnp.int32)
    s1 = jnp.searchsorted(b1, vals).astype(jnp.int32)
    s2 = jnp.searchsorted(b2, vals).astype(jnp.int32)
    pad = jnp.full((L - 9,), n, jnp.int32)
    nbrow = jnp.full((L,), b1[-1], jnp.int32)
    aux = jnp.concatenate([s1, pad, s2, pad, nbrow])
    mv = mean.reshape(-1)
    vv = variance.reshape(-1)
    fn = _make_sc_chamfer(n, mv.shape[0], interpret=interpret)
    partial = fn(p1[:, 0], p1[:, 1], p1[:, 2], b1,
                 p2[:, 0], p2[:, 1], p2[:, 2], b2, aux, mv, vv)
    return jnp.sum(partial)


def kernel(in_points_list, in_batch_list, out_points_list, out_batch_list,
           mean, variance):
    return _run(in_points_list[0], in_batch_list[0],
                out_points_list[0], out_batch_list[0], mean, variance)
